# Initial kernel scaffold; baseline (speedup 1.0000x reference)
import jax
import jax.numpy as jnp
from jax.experimental import pallas as pl


def kernel(rt_pred, rt_true):
    raise NotImplementedError("write your pallas kernel here")


# speedup vs baseline: 40.6191x; 40.6191x over previous
import functools
import math

import jax
import jax.numpy as jnp
from jax import lax
from jax.experimental import pallas as pl
from jax.experimental.pallas import tpu as pltpu
from jax.experimental.pallas import tpu_sc as plsc

N = 4194304
MAXS = 512
NBINS = MAXS + 1
BPAD = 520
LANES = 16
NC, NS = 2, 16
NW = NC * NS
PER_W = N // NW
CHUNK = 16384
NCHUNK = PER_W // CHUNK
VPC = CHUNK // LANES
HSIZE = LANES * BPAD

_mesh = plsc.VectorSubcoreMesh(core_axis_name="c", subcore_axis_name="s")


@functools.partial(
    pl.kernel,
    out_type=(
        jax.ShapeDtypeStruct((NW, HSIZE), jnp.float32),
        jax.ShapeDtypeStruct((NW * 2 * LANES,), jnp.float32),
    ),
    mesh=_mesh,
    scratch_types=(
        pltpu.VMEM((2, CHUNK), jnp.int32),
        pltpu.VMEM((2, CHUNK), jnp.float32),
        pltpu.VMEM((HSIZE,), jnp.float32),
        pltpu.VMEM((32,), jnp.float32),
        pltpu.SemaphoreType.DMA,
        pltpu.SemaphoreType.DMA,
    ),
    compiler_params=pltpu.CompilerParams(needs_layout_passes=False),
)
def _sc_stats(pred_hbm, true_hbm, hist_out, mom_out,
              pred_v, true_v, hist_v, mom_v, sem_p, sem_t):
    wid = lax.axis_index("s") * NC + lax.axis_index("c")
    base = wid * PER_W

    lane_off = lax.iota(jnp.int32, LANES) * BPAD
    ones = jnp.full((LANES,), 1.0, jnp.float32)
    zeros = jnp.zeros((LANES,), jnp.float32)

    def _zero(j, carry):
        hist_v[pl.ds(j * LANES, LANES)] = zeros
        return carry
    lax.fori_loop(0, HSIZE // LANES, _zero, 0)

    def _chunk_body(buf, carry):
        def _it(v, c):
            sv, qv = c
            p = pred_v[buf, pl.ds(v * LANES, LANES)]
            plsc.addupdate_scatter(hist_v, [p + lane_off], ones)
            t = true_v[buf, pl.ds(v * LANES, LANES)]
            return (sv + t, qv + t * t)
        return lax.fori_loop(0, VPC, _it, carry)

    cps = [None, None]
    cps[0] = (
        pltpu.async_copy(pred_hbm.at[pl.ds(base, CHUNK)], pred_v.at[0], sem_p),
        pltpu.async_copy(true_hbm.at[pl.ds(base, CHUNK)], true_v.at[0], sem_t),
    )
    carry = (zeros, zeros)
    for c in range(NCHUNK):
        nb = (c + 1) % 2
        if c + 1 < NCHUNK:
            off = base + (c + 1) * CHUNK
            cps[nb] = (
                pltpu.async_copy(pred_hbm.at[pl.ds(off, CHUNK)], pred_v.at[nb], sem_p),
                pltpu.async_copy(true_hbm.at[pl.ds(off, CHUNK)], true_v.at[nb], sem_t),
            )
        cb = c % 2
        cps[cb][0].wait()
        cps[cb][1].wait()
        carry = _chunk_body(cb, carry)

    sv, qv = carry
    mom_v[pl.ds(0, LANES)] = sv
    mom_v[pl.ds(LANES, LANES)] = qv
    pltpu.sync_copy(hist_v, hist_out.at[wid])
    pltpu.sync_copy(mom_v, mom_out.at[pl.ds(wid * 2 * LANES, 2 * LANES)])


def _loss_body(hp_ref, sp_ref, qp_ref, out_ref):
    n = jnp.float32(N)
    s = jnp.sum(sp_ref[...])
    q = jnp.sum(qp_ref[...])
    mu = s / n
    var = (q - s * mu) / (n - 1.0)
    sigma = jnp.sqrt(var)
    hist = jnp.sum(hp_ref[...], axis=0, keepdims=True)
    xi = lax.broadcasted_iota(jnp.int32, (1, BPAD), 1)
    xs = xi.astype(jnp.float32)
    mask = xi < NBINS
    z = (xs - mu) / sigma
    logp = -0.5 * z * z - jnp.log(sigma) - jnp.float32(0.5 * math.log(2.0 * math.pi))
    d = jnp.where(mask, jnp.exp(logp), 0.0)
    denom = jnp.maximum(jnp.sum(jnp.abs(hist)), 1e-12)
    pdist = hist / denom
    pw = jnp.where(mask, jnp.exp(d) * (d - pdist), 0.0)
    out_ref[...] = jnp.reshape(jnp.sum(pw) / jnp.float32(NBINS), (1, 1))


_tc_loss = pl.pallas_call(
    _loss_body,
    out_shape=jax.ShapeDtypeStruct((1, 1), jnp.float32),
)


def kernel(rt_pred, rt_true):
    hp, mom = _sc_stats(rt_pred, rt_true)
    mom = mom.reshape(NW, 2, LANES)
    out = _tc_loss(hp.reshape(NW * LANES, BPAD), mom[:, 0, :], mom[:, 1, :])
    return out[0, 0]


# speedup vs baseline: 67.7171x; 1.6671x over previous
import functools
import math

import jax
import jax.numpy as jnp
from jax import lax
from jax.experimental import pallas as pl
from jax.experimental.pallas import tpu as pltpu
from jax.experimental.pallas import tpu_sc as plsc

N = 4194304
MAXS = 512
NBINS = MAXS + 1
BPAD = 520
LANES = 16
NC, NS = 2, 16
NW = NC * NS
PER_W = N // NW
CHUNK = 16384
NCHUNK = PER_W // CHUNK
VPC = CHUNK // LANES
HSIZE = LANES * BPAD

_mesh = plsc.VectorSubcoreMesh(core_axis_name="c", subcore_axis_name="s")


@functools.partial(
    pl.kernel,
    out_type=(
        jax.ShapeDtypeStruct((NW, HSIZE), jnp.float32),
        jax.ShapeDtypeStruct((NW * 2 * LANES,), jnp.float32),
    ),
    mesh=_mesh,
    scratch_types=(
        pltpu.VMEM((2, CHUNK), jnp.int32),
        pltpu.VMEM((2, CHUNK), jnp.float32),
        pltpu.VMEM((HSIZE,), jnp.float32),
        pltpu.VMEM((32,), jnp.float32),
        pltpu.SemaphoreType.DMA,
        pltpu.SemaphoreType.DMA,
    ),
    compiler_params=pltpu.CompilerParams(needs_layout_passes=False),
)
def _sc_stats(pred_hbm, true_hbm, hist_out, mom_out,
              pred_v, true_v, hist_v, mom_v, sem_p, sem_t):
    wid = lax.axis_index("s") * NC + lax.axis_index("c")
    base = wid * PER_W

    lane_off = lax.iota(jnp.int32, LANES) * BPAD
    ones = jnp.full((LANES,), 1.0, jnp.float32)
    zeros = jnp.zeros((LANES,), jnp.float32)

    def _zero(j, carry):
        hist_v[pl.ds(j * LANES, LANES)] = zeros
        return carry
    lax.fori_loop(0, HSIZE // LANES, _zero, 0)

    def _chunk_body(buf, carry):
        @plsc.parallel_loop(0, VPC, step=1, unroll=8, carry=carry)
        def _it(v, c):
            sv, qv = c
            p = pred_v[buf, pl.ds(v * LANES, LANES)]
            plsc.addupdate_scatter(hist_v, [p + lane_off], ones)
            t = true_v[buf, pl.ds(v * LANES, LANES)]
            return (sv + t, qv + t * t)
        return _it

    cps = [None, None]
    cps[0] = (
        pltpu.async_copy(pred_hbm.at[pl.ds(base, CHUNK)], pred_v.at[0], sem_p),
        pltpu.async_copy(true_hbm.at[pl.ds(base, CHUNK)], true_v.at[0], sem_t),
    )
    carry = (zeros, zeros)
    for c in range(NCHUNK):
        nb = (c + 1) % 2
        if c + 1 < NCHUNK:
            off = base + (c + 1) * CHUNK
            cps[nb] = (
                pltpu.async_copy(pred_hbm.at[pl.ds(off, CHUNK)], pred_v.at[nb], sem_p),
                pltpu.async_copy(true_hbm.at[pl.ds(off, CHUNK)], true_v.at[nb], sem_t),
            )
        cb = c % 2
        cps[cb][0].wait()
        cps[cb][1].wait()
        carry = _chunk_body(cb, carry)

    sv, qv = carry
    mom_v[pl.ds(0, LANES)] = sv
    mom_v[pl.ds(LANES, LANES)] = qv
    pltpu.sync_copy(hist_v, hist_out.at[wid])
    pltpu.sync_copy(mom_v, mom_out.at[pl.ds(wid * 2 * LANES, 2 * LANES)])


def _loss_body(hp_ref, sp_ref, qp_ref, out_ref):
    n = jnp.float32(N)
    s = jnp.sum(sp_ref[...])
    q = jnp.sum(qp_ref[...])
    mu = s / n
    var = (q - s * mu) / (n - 1.0)
    sigma = jnp.sqrt(var)
    hist = jnp.sum(hp_ref[...], axis=0, keepdims=True)
    xi = lax.broadcasted_iota(jnp.int32, (1, BPAD), 1)
    xs = xi.astype(jnp.float32)
    mask = xi < NBINS
    z = (xs - mu) / sigma
    logp = -0.5 * z * z - jnp.log(sigma) - jnp.float32(0.5 * math.log(2.0 * math.pi))
    d = jnp.where(mask, jnp.exp(logp), 0.0)
    denom = jnp.maximum(jnp.sum(jnp.abs(hist)), 1e-12)
    pdist = hist / denom
    pw = jnp.where(mask, jnp.exp(d) * (d - pdist), 0.0)
    out_ref[...] = jnp.reshape(jnp.sum(pw) / jnp.float32(NBINS), (1, 1))


_tc_loss = pl.pallas_call(
    _loss_body,
    out_shape=jax.ShapeDtypeStruct((1, 1), jnp.float32),
)


def kernel(rt_pred, rt_true):
    hp, mom = _sc_stats(rt_pred, rt_true)
    mom = mom.reshape(NW, 2, LANES)
    out = _tc_loss(hp.reshape(NW * LANES, BPAD), mom[:, 0, :], mom[:, 1, :])
    return out[0, 0]
